# Initial kernel scaffold; baseline (speedup 1.0000x reference)
import jax
import jax.numpy as jnp
from jax.experimental import pallas as pl


def kernel(x, edge_index, W1, as1, ad1, b1, W2, as2, ad2, b2, W3, as3, ad3, b3, fcW, fcb, outW, outb):
    raise NotImplementedError("write your pallas kernel here")


# speedup vs baseline: 11.5103x; 11.5103x over previous
import functools

import jax
import jax.numpy as jnp
from jax import lax
from jax.experimental import pallas as pl
from jax.experimental.pallas import tpu as pltpu
from jax.experimental.pallas import tpu_sc as plsc

N = 10000
D = 128
F = 256
NPAD = 10240
NT = 16
NC = 2
KROWS = 84
EPW = KROWS * 128
E = 160000 + N
EV = E // NT
SW = NPAD // NT
MB = 512


def _tc1_body(x_ref, w_ref, as_ref, ad_ref, hlo_ref, hhi_ref, av_ref, bv_ref):
    h = jnp.dot(x_ref[...], w_ref[...], preferred_element_type=jnp.float32)
    hlo_ref[...] = h[:, :128]
    hhi_ref[...] = h[:, 128:]
    av_ref[...] = jnp.dot(h, as_ref[...], preferred_element_type=jnp.float32)
    bv_ref[...] = jnp.dot(h, ad_ref[...], preferred_element_type=jnp.float32)


def _tc2_body(lo_ref, hi_ref, den_ref, b3lo_ref, b3hi_ref, fwlo_ref, fwhi_ref,
              fb_ref, ow_ref, ob_ref, out_ref):
    d = den_ref[...] + 1e-16
    lo = jnp.maximum(lo_ref[...] / d + b3lo_ref[...], 0.0)
    hi = jnp.maximum(hi_ref[...] / d + b3hi_ref[...], 0.0)
    g = jnp.dot(lo, fwlo_ref[...], preferred_element_type=jnp.float32)
    g = g + jnp.dot(hi, fwhi_ref[...], preferred_element_type=jnp.float32)
    g = jnp.maximum(g + fb_ref[...], 0.0)
    out_ref[...] = jnp.dot(g, ow_ref[...], preferred_element_type=jnp.float32) + ob_ref[...]


def _sc_body(src_hbm, dst_hbm, as_hbm, ad_hbm, hlo_hbm, hhi_hbm,
             acc_out, den_out,
             src_v, dst_v, asrow_v, adrow_v, exrow_v, rows_v, zden_v,
             acc_sh, den_sh):
    cid = lax.axis_index("c")
    sid = lax.axis_index("s")
    z16 = jnp.zeros((16,), jnp.float32)

    def _zb(i, c):
        rows_v[i // 8, pl.ds((i % 8) * 16, 16)] = z16
        return c
    lax.fori_loop(0, 128 * 8, _zb, 0)

    def _zd(i, c):
        zden_v[pl.ds(i * 16, 16)] = z16
        return c
    lax.fori_loop(0, SW // 16, _zd, 0)

    pltpu.sync_copy(src_hbm.at[sid], src_v)
    pltpu.sync_copy(dst_hbm.at[sid], dst_v)

    def _za(i, c):
        pltpu.sync_copy(rows_v, acc_sh.at[pl.ds(sid * SW + i * 128, 128)])
        return c
    lax.fori_loop(0, SW // 128, _za, 0)
    pltpu.sync_copy(zden_v, den_sh.at[pl.ds(sid * SW, SW)])
    plsc.subcore_barrier()

    iota = lax.iota(jnp.int32, 16)

    def _edge_batch(k, carry):
        pltpu.sync_copy(as_hbm.at[src_v.at[k]], asrow_v)
        pltpu.sync_copy(ad_hbm.at[dst_v.at[k]], adrow_v)
        for c in range(8):
            av = asrow_v[pl.ds(c * 16, 16)]
            bv = adrow_v[pl.ds(c * 16, 16)]
            e = av + bv
            e = jnp.where(e > 0.0, e, 0.2 * e)
            ex = jnp.exp(e)
            gi = k * 128 + c * 16 + iota
            ex = jnp.where(gi < EV, ex, 0.0)
            exrow_v[pl.ds(c * 16, 16)] = ex

        @pl.when(cid == 0)
        def _():
            pltpu.sync_copy(hlo_hbm.at[src_v.at[k]], rows_v)

        @pl.when(cid == 1)
        def _():
            pltpu.sync_copy(hhi_hbm.at[src_v.at[k]], rows_v)

        def _scale(r, c2):
            ab = plsc.load_gather(exrow_v, [jnp.full((16,), r, jnp.int32)])
            for cc in range(8):
                rows_v[r, pl.ds(cc * 16, 16)] = rows_v[r, pl.ds(cc * 16, 16)] * ab
            return c2
        lax.fori_loop(0, 128, _scale, 0)

        pltpu.sync_copy(rows_v, acc_sh.at[dst_v.at[k]], add=True)
        pltpu.sync_copy(exrow_v, den_sh.at[dst_v.at[k]], add=True)
        return carry

    lax.fori_loop(0, KROWS, _edge_batch, 0)
    plsc.subcore_barrier()

    pltpu.sync_copy(acc_sh.at[pl.ds(sid * SW, SW)],
                    acc_out.at[cid].at[pl.ds(sid * SW, SW)])
    pltpu.sync_copy(den_sh.at[pl.ds(sid * SW, SW)],
                    den_out.at[cid].at[pl.ds(sid * SW, SW)])


_sc_call = functools.partial(
    pl.kernel,
    out_type=(
        jax.ShapeDtypeStruct((NC, NPAD, 128), jnp.float32),
        jax.ShapeDtypeStruct((NC, NPAD), jnp.float32),
    ),
    mesh=plsc.VectorSubcoreMesh(core_axis_name="c", subcore_axis_name="s",
                                num_cores=NC, num_subcores=NT),
    compiler_params=pltpu.CompilerParams(needs_layout_passes=False),
    scratch_types=[
        pltpu.VMEM((KROWS, 128), jnp.int32),
        pltpu.VMEM((KROWS, 128), jnp.int32),
        pltpu.VMEM((128,), jnp.float32),
        pltpu.VMEM((128,), jnp.float32),
        pltpu.VMEM((128,), jnp.float32),
        pltpu.VMEM((128, 128), jnp.float32),
        pltpu.VMEM((SW,), jnp.float32),
        pltpu.VMEM_SHARED((NPAD, 128), jnp.float32),
        pltpu.VMEM_SHARED((NPAD,), jnp.float32),
    ],
)


def kernel(x, edge_index, W1, as1, ad1, b1, W2, as2, ad2, b2, W3, as3, ad3, b3,
           fcW, fcb, outW, outb):
    f32 = jnp.float32
    x_pad = jnp.zeros((NPAD, D), f32).at[:N].set(x)

    loops = jnp.arange(N, dtype=edge_index.dtype)
    srcf = jnp.concatenate([edge_index[0], loops])
    dstf = jnp.concatenate([edge_index[1], loops])
    src3 = jnp.zeros((NT, EPW), jnp.int32).at[:, :EV].set(srcf.reshape(NT, EV))
    dst3 = jnp.zeros((NT, EPW), jnp.int32).at[:, :EV].set(dstf.reshape(NT, EV))
    src3 = src3.reshape(NT, KROWS, 128)
    dst3 = dst3.reshape(NT, KROWS, 128)

    grid = (NPAD // MB,)
    hlo, hhi, a_s, a_d = pl.pallas_call(
        _tc1_body,
        grid=grid,
        in_specs=[
            pl.BlockSpec((MB, D), lambda i: (i, 0)),
            pl.BlockSpec((D, F), lambda i: (0, 0)),
            pl.BlockSpec((F, 1), lambda i: (0, 0)),
            pl.BlockSpec((F, 1), lambda i: (0, 0)),
        ],
        out_specs=[
            pl.BlockSpec((MB, 128), lambda i: (i, 0)),
            pl.BlockSpec((MB, 128), lambda i: (i, 0)),
            pl.BlockSpec((MB, 1), lambda i: (i, 0)),
            pl.BlockSpec((MB, 1), lambda i: (i, 0)),
        ],
        out_shape=[
            jax.ShapeDtypeStruct((NPAD, 128), f32),
            jax.ShapeDtypeStruct((NPAD, 128), f32),
            jax.ShapeDtypeStruct((NPAD, 1), f32),
            jax.ShapeDtypeStruct((NPAD, 1), f32),
        ],
    )(x_pad, W3, as3.reshape(F, 1), ad3.reshape(F, 1))

    acc, den = _sc_call(_sc_body)(
        src3, dst3, a_s.reshape(NPAD), a_d.reshape(NPAD), hlo, hhi)

    out_pad = pl.pallas_call(
        _tc2_body,
        grid=grid,
        in_specs=[
            pl.BlockSpec((MB, 128), lambda i: (i, 0)),
            pl.BlockSpec((MB, 128), lambda i: (i, 0)),
            pl.BlockSpec((MB, 1), lambda i: (i, 0)),
            pl.BlockSpec((1, 128), lambda i: (0, 0)),
            pl.BlockSpec((1, 128), lambda i: (0, 0)),
            pl.BlockSpec((128, 128), lambda i: (0, 0)),
            pl.BlockSpec((128, 128), lambda i: (0, 0)),
            pl.BlockSpec((1, 128), lambda i: (0, 0)),
            pl.BlockSpec((128, 128), lambda i: (0, 0)),
            pl.BlockSpec((1, 128), lambda i: (0, 0)),
        ],
        out_specs=pl.BlockSpec((MB, 128), lambda i: (i, 0)),
        out_shape=jax.ShapeDtypeStruct((NPAD, 128), f32),
    )(
        acc[0], acc[1], den[0].reshape(NPAD, 1),
        b3[:128].reshape(1, 128), b3[128:].reshape(1, 128),
        fcW[:128], fcW[128:], fcb.reshape(1, 128),
        jnp.zeros((128, 128), f32).at[:, :6].set(outW),
        jnp.zeros((1, 128), f32).at[0, :6].set(outb),
    )
    return out_pad[:N, :6]


# speedup vs baseline: 19.1472x; 1.6635x over previous
import functools

import jax
import jax.numpy as jnp
from jax import lax
from jax.experimental import pallas as pl
from jax.experimental.pallas import tpu as pltpu
from jax.experimental.pallas import tpu_sc as plsc

N = 10000
D = 128
F = 256
NPAD = 10240
NT = 16
NC = 2
KROWS = 84
EPW = KROWS * 128
E = 160000 + N
EV = E // NT
SW = NPAD // NT
MB = 512


def _tc1_body(x_ref, w_ref, as_ref, ad_ref, hlo_ref, hhi_ref, av_ref, bv_ref):
    h = jnp.dot(x_ref[...], w_ref[...], preferred_element_type=jnp.float32)
    hlo_ref[...] = h[:, :128]
    hhi_ref[...] = h[:, 128:]
    av_ref[...] = jnp.dot(h, as_ref[...], preferred_element_type=jnp.float32)
    bv_ref[...] = jnp.dot(h, ad_ref[...], preferred_element_type=jnp.float32)


def _tc2_body(lo_ref, hi_ref, den_ref, b3lo_ref, b3hi_ref, fwlo_ref, fwhi_ref,
              fb_ref, ow_ref, ob_ref, out_ref):
    d = den_ref[...] + 1e-16
    lo = jnp.maximum(lo_ref[...] / d + b3lo_ref[...], 0.0)
    hi = jnp.maximum(hi_ref[...] / d + b3hi_ref[...], 0.0)
    g = jnp.dot(lo, fwlo_ref[...], preferred_element_type=jnp.float32)
    g = g + jnp.dot(hi, fwhi_ref[...], preferred_element_type=jnp.float32)
    g = jnp.maximum(g + fb_ref[...], 0.0)
    out_ref[...] = jnp.dot(g, ow_ref[...], preferred_element_type=jnp.float32) + ob_ref[...]


def _sc_body(src_hbm, dst_hbm, as_hbm, ad_hbm, hlo_hbm, hhi_hbm,
             acc_out, den_out,
             esrc_v, edst_v, asrow_v, adrow_v, exrow_v, rows_v,
             acc_sh, den_sh, esem, gsem, ssem):
    cid = lax.axis_index("c")
    sid = lax.axis_index("s")
    z16 = jnp.zeros((16,), jnp.float32)

    def _zb(i, c):
        rows_v[0, i // 8, pl.ds((i % 8) * 16, 16)] = z16
        return c
    lax.fori_loop(0, 128 * 8, _zb, 0)
    for c in range(8):
        asrow_v[0, pl.ds(c * 16, 16)] = z16

    def _za(i, c):
        pltpu.sync_copy(rows_v.at[0], acc_sh.at[pl.ds(sid * SW + i * 128, 128)])
        pltpu.sync_copy(asrow_v.at[0], den_sh.at[pl.ds(sid * SW + i * 128, 128)])
        return c
    lax.fori_loop(0, SW // 128, _za, 0)
    plsc.subcore_barrier()

    iota = lax.iota(jnp.int32, 16)

    def _issue_efetch(k):
        j = jnp.bitwise_and(k, 3)
        pltpu.async_copy(src_hbm.at[sid].at[k], esrc_v.at[j], esem.at[j])
        pltpu.async_copy(dst_hbm.at[sid].at[k], edst_v.at[j], esem.at[j])

    def _wait_efetch(k):
        j = jnp.bitwise_and(k, 3)
        pltpu.make_async_copy(src_hbm.at[sid].at[k], esrc_v.at[j], esem.at[j]).wait()
        pltpu.make_async_copy(dst_hbm.at[sid].at[k], edst_v.at[j], esem.at[j]).wait()

    def _issue_gather(k, b):
        j = jnp.bitwise_and(k, 3)

        @pl.when(cid == 0)
        def _():
            pltpu.async_copy(hlo_hbm.at[esrc_v.at[j]], rows_v.at[b], gsem.at[b])

        @pl.when(cid == 1)
        def _():
            pltpu.async_copy(hhi_hbm.at[esrc_v.at[j]], rows_v.at[b], gsem.at[b])

        pltpu.async_copy(as_hbm.at[esrc_v.at[j]], asrow_v.at[b], gsem.at[b])
        pltpu.async_copy(ad_hbm.at[edst_v.at[j]], adrow_v.at[b], gsem.at[b])

    def _wait_gather(k, b):
        j = jnp.bitwise_and(k, 3)
        pltpu.make_async_copy(hlo_hbm.at[esrc_v.at[j]], rows_v.at[b], gsem.at[b]).wait()
        pltpu.make_async_copy(as_hbm.at[esrc_v.at[j]], asrow_v.at[b], gsem.at[b]).wait()
        pltpu.make_async_copy(ad_hbm.at[edst_v.at[j]], adrow_v.at[b], gsem.at[b]).wait()

    def _issue_scatter(k, b):
        j = jnp.bitwise_and(k, 3)
        pltpu.async_copy(rows_v.at[b], acc_sh.at[edst_v.at[j]], ssem.at[b], add=True)
        pltpu.async_copy(exrow_v.at[b], den_sh.at[edst_v.at[j]], ssem.at[b], add=True)

    def _wait_scatter(k, b):
        j = jnp.bitwise_and(k, 3)
        pltpu.make_async_copy(rows_v.at[b], acc_sh.at[edst_v.at[j]], ssem.at[b]).wait()
        pltpu.make_async_copy(exrow_v.at[b], den_sh.at[edst_v.at[j]], ssem.at[b]).wait()

    _issue_efetch(jnp.int32(0))
    _issue_efetch(jnp.int32(1))
    _issue_efetch(jnp.int32(2))
    _wait_efetch(jnp.int32(0))
    _issue_gather(jnp.int32(0), jnp.int32(0))

    def _edge_batch(k, carry):
        p = jnp.bitwise_and(k, 1)
        q = 1 - p
        _wait_gather(k, p)

        @pl.when(k >= 1)
        def _():
            _wait_scatter(k - 1, q)

        @pl.when(k + 1 < KROWS)
        def _():
            _wait_efetch(k + 1)
            _issue_gather(k + 1, q)

        @pl.when(k + 3 < KROWS)
        def _():
            _issue_efetch(k + 3)

        for c in range(8):
            av = asrow_v[p, pl.ds(c * 16, 16)]
            bv = adrow_v[p, pl.ds(c * 16, 16)]
            e = av + bv
            e = jnp.where(e > 0.0, e, 0.2 * e)
            ex = jnp.exp(e)
            gi = k * 128 + c * 16 + iota
            ex = jnp.where(gi < EV, ex, 0.0)
            exrow_v[p, pl.ds(c * 16, 16)] = ex

        def _scale(r, c2):
            ab = plsc.load_gather(exrow_v.at[p], [jnp.full((16,), r, jnp.int32)])
            for cc in range(8):
                rows_v[p, r, pl.ds(cc * 16, 16)] = rows_v[p, r, pl.ds(cc * 16, 16)] * ab
            return c2
        lax.fori_loop(0, 128, _scale, 0)

        _issue_scatter(k, p)
        return carry

    lax.fori_loop(0, KROWS, _edge_batch, 0)
    _wait_scatter(jnp.int32(KROWS - 1), jnp.int32((KROWS - 1) & 1))
    plsc.subcore_barrier()

    pltpu.sync_copy(acc_sh.at[pl.ds(sid * SW, SW)],
                    acc_out.at[cid].at[pl.ds(sid * SW, SW)])
    pltpu.sync_copy(den_sh.at[pl.ds(sid * SW, SW)],
                    den_out.at[cid].at[pl.ds(sid * SW, SW)])


_sc_call = functools.partial(
    pl.kernel,
    out_type=(
        jax.ShapeDtypeStruct((NC, NPAD, 128), jnp.float32),
        jax.ShapeDtypeStruct((NC, NPAD), jnp.float32),
    ),
    mesh=plsc.VectorSubcoreMesh(core_axis_name="c", subcore_axis_name="s",
                                num_cores=NC, num_subcores=NT),
    compiler_params=pltpu.CompilerParams(needs_layout_passes=False),
    scratch_types=[
        pltpu.VMEM((4, 128), jnp.int32),
        pltpu.VMEM((4, 128), jnp.int32),
        pltpu.VMEM((2, 128), jnp.float32),
        pltpu.VMEM((2, 128), jnp.float32),
        pltpu.VMEM((2, 128), jnp.float32),
        pltpu.VMEM((2, 128, 128), jnp.float32),
        pltpu.VMEM_SHARED((NPAD, 128), jnp.float32),
        pltpu.VMEM_SHARED((NPAD,), jnp.float32),
        pltpu.SemaphoreType.DMA((4,)),
        pltpu.SemaphoreType.DMA((2,)),
        pltpu.SemaphoreType.DMA((2,)),
    ],
)


def kernel(x, edge_index, W1, as1, ad1, b1, W2, as2, ad2, b2, W3, as3, ad3, b3,
           fcW, fcb, outW, outb):
    f32 = jnp.float32
    x_pad = jnp.zeros((NPAD, D), f32).at[:N].set(x)

    loops = jnp.arange(N, dtype=edge_index.dtype)
    srcf = jnp.concatenate([edge_index[0], loops])
    dstf = jnp.concatenate([edge_index[1], loops])
    src3 = jnp.zeros((NT, EPW), jnp.int32).at[:, :EV].set(srcf.reshape(NT, EV))
    dst3 = jnp.zeros((NT, EPW), jnp.int32).at[:, :EV].set(dstf.reshape(NT, EV))
    src3 = src3.reshape(NT, KROWS, 128)
    dst3 = dst3.reshape(NT, KROWS, 128)

    grid = (NPAD // MB,)
    hlo, hhi, a_s, a_d = pl.pallas_call(
        _tc1_body,
        grid=grid,
        in_specs=[
            pl.BlockSpec((MB, D), lambda i: (i, 0)),
            pl.BlockSpec((D, F), lambda i: (0, 0)),
            pl.BlockSpec((F, 1), lambda i: (0, 0)),
            pl.BlockSpec((F, 1), lambda i: (0, 0)),
        ],
        out_specs=[
            pl.BlockSpec((MB, 128), lambda i: (i, 0)),
            pl.BlockSpec((MB, 128), lambda i: (i, 0)),
            pl.BlockSpec((MB, 1), lambda i: (i, 0)),
            pl.BlockSpec((MB, 1), lambda i: (i, 0)),
        ],
        out_shape=[
            jax.ShapeDtypeStruct((NPAD, 128), f32),
            jax.ShapeDtypeStruct((NPAD, 128), f32),
            jax.ShapeDtypeStruct((NPAD, 1), f32),
            jax.ShapeDtypeStruct((NPAD, 1), f32),
        ],
    )(x_pad, W3, as3.reshape(F, 1), ad3.reshape(F, 1))

    acc, den = _sc_call(_sc_body)(
        src3, dst3, a_s.reshape(NPAD), a_d.reshape(NPAD), hlo, hhi)

    out_pad = pl.pallas_call(
        _tc2_body,
        grid=grid,
        in_specs=[
            pl.BlockSpec((MB, 128), lambda i: (i, 0)),
            pl.BlockSpec((MB, 128), lambda i: (i, 0)),
            pl.BlockSpec((MB, 1), lambda i: (i, 0)),
            pl.BlockSpec((1, 128), lambda i: (0, 0)),
            pl.BlockSpec((1, 128), lambda i: (0, 0)),
            pl.BlockSpec((128, 128), lambda i: (0, 0)),
            pl.BlockSpec((128, 128), lambda i: (0, 0)),
            pl.BlockSpec((1, 128), lambda i: (0, 0)),
            pl.BlockSpec((128, 128), lambda i: (0, 0)),
            pl.BlockSpec((1, 128), lambda i: (0, 0)),
        ],
        out_specs=pl.BlockSpec((MB, 128), lambda i: (i, 0)),
        out_shape=jax.ShapeDtypeStruct((NPAD, 128), f32),
    )(
        acc[0], acc[1], den[0].reshape(NPAD, 1),
        b3[:128].reshape(1, 128), b3[128:].reshape(1, 128),
        fcW[:128], fcW[128:], fcb.reshape(1, 128),
        jnp.zeros((128, 128), f32).at[:, :6].set(outW),
        jnp.zeros((1, 128), f32).at[0, :6].set(outb),
    )
    return out_pad[:N, :6]
